# baseline (speedup 1.0000x reference)
import jax
import jax.numpy as jnp
from jax.experimental import pallas as pl

_INPUT_LEN = 200
_EMBED_DIM = 64
_BATCH = 4096
_BB = 128


def _bcast_body(pos_ref, out_ref):
    out_ref[...] = jnp.broadcast_to(pos_ref[...][None], out_ref.shape)


def kernel(x, pos_table):
    del x
    return pl.pallas_call(
        _bcast_body,
        grid=(_BATCH // _BB,),
        in_specs=[pl.BlockSpec((_INPUT_LEN, _EMBED_DIM), lambda i: (0, 0))],
        out_specs=pl.BlockSpec((_BB, _INPUT_LEN, _EMBED_DIM), lambda i: (i, 0, 0)),
        out_shape=jax.ShapeDtypeStruct((_BATCH, _INPUT_LEN, _EMBED_DIM), jnp.float32),
    )(pos_table)
